# Initial kernel scaffold; baseline (speedup 1.0000x reference)
import jax
import jax.numpy as jnp
from jax.experimental import pallas as pl


def kernel(x, wt, b2d):
    raise NotImplementedError("write your pallas kernel here")


# speedup vs baseline: 3.0449x; 3.0449x over previous
import jax
import jax.numpy as jnp
from jax.experimental import pallas as pl
from jax.experimental.pallas import tpu as pltpu


def _matmul_bias_kernel(x_ref, w_ref, b_ref, o_ref):
    xb = x_ref[...].astype(jnp.bfloat16)
    acc = jnp.dot(xb, w_ref[...], preferred_element_type=jnp.float32)
    o_ref[...] = acc + b_ref[...]


def kernel(x, wt, b2d):
    n_out = wt.shape[1]
    *lead, K = x.shape
    x2d = x.reshape(-1, K)
    M = x2d.shape[0]
    Kp, Np = wt.shape

    w_bf = wt.astype(jnp.bfloat16)

    tm = 512
    Mp = (M + tm - 1) // tm * tm
    if Mp != M or Kp != K:
        x2d = jnp.pad(x2d, ((0, Mp - M), (0, Kp - K)))

    grid = (Mp // tm,)

    cost = pl.CostEstimate(
        flops=2 * Mp * Np * Kp,
        transcendentals=0,
        bytes_accessed=Mp * Kp * 4 + Kp * Np * 2 + Np * 4 + Mp * Np * 4,
    )

    out = pl.pallas_call(
        _matmul_bias_kernel,
        out_shape=jax.ShapeDtypeStruct((Mp, Np), x.dtype),
        grid=grid,
        in_specs=[
            pl.BlockSpec((tm, Kp), lambda i: (i, 0)),
            pl.BlockSpec((Kp, Np), lambda i: (0, 0)),
            pl.BlockSpec((1, Np), lambda i: (0, 0)),
        ],
        out_specs=pl.BlockSpec((tm, Np), lambda i: (i, 0)),
        compiler_params=pltpu.CompilerParams(
            dimension_semantics=("parallel",),
            vmem_limit_bytes=64 * 1024 * 1024,
        ),
        cost_estimate=cost,
    )(x2d, w_bf, b2d)

    if Mp != M or Np != n_out:
        out = out[:M, :n_out]
    return out.reshape(*lead, n_out)


# speedup vs baseline: 3.2527x; 1.0682x over previous
import jax
import jax.numpy as jnp
from jax.experimental import pallas as pl
from jax.experimental.pallas import tpu as pltpu


def _matmul_bias_kernel(x_ref, w_ref, b_ref, o_ref, w_bf_ref):
    @pl.when(pl.program_id(0) == 0)
    def _cast_weight():
        w_bf_ref[...] = w_ref[...].astype(jnp.bfloat16)

    xb = x_ref[...].astype(jnp.bfloat16)
    acc = jnp.dot(xb, w_bf_ref[...], preferred_element_type=jnp.float32)
    o_ref[...] = acc + b_ref[...]


def kernel(x, wt, b2d):
    n_out = wt.shape[1]
    *lead, K = x.shape
    x2d = x.reshape(-1, K)
    M = x2d.shape[0]
    Kp, Np = wt.shape

    tm = 1024
    Mp = (M + tm - 1) // tm * tm
    if Mp != M or Kp != K:
        x2d = jnp.pad(x2d, ((0, Mp - M), (0, Kp - K)))

    grid = (Mp // tm,)

    cost = pl.CostEstimate(
        flops=2 * Mp * Np * Kp,
        transcendentals=0,
        bytes_accessed=Mp * Kp * 4 + Kp * Np * 4 + Np * 4 + Mp * Np * 4,
    )

    out = pl.pallas_call(
        _matmul_bias_kernel,
        out_shape=jax.ShapeDtypeStruct((Mp, Np), x.dtype),
        grid=grid,
        in_specs=[
            pl.BlockSpec((tm, Kp), lambda i: (i, 0)),
            pl.BlockSpec((Kp, Np), lambda i: (0, 0)),
            pl.BlockSpec((1, Np), lambda i: (0, 0)),
        ],
        out_specs=pl.BlockSpec((tm, Np), lambda i: (i, 0)),
        scratch_shapes=[pltpu.VMEM((Kp, Np), jnp.bfloat16)],
        compiler_params=pltpu.CompilerParams(
            dimension_semantics=("arbitrary",),
            vmem_limit_bytes=100 * 1024 * 1024,
        ),
        cost_estimate=cost,
    )(x2d, wt, b2d)

    if Mp != M or Np != n_out:
        out = out[:M, :n_out]
    return out.reshape(*lead, n_out)


# speedup vs baseline: 3.2789x; 1.0081x over previous
import jax
import jax.numpy as jnp
from jax.experimental import pallas as pl
from jax.experimental.pallas import tpu as pltpu


def _matmul_bias_kernel(x_ref, w_ref, b_ref, o_ref, w_bf_ref):
    @pl.when(pl.program_id(0) == 0)
    def _cast_weight():
        w_bf_ref[...] = w_ref[...].astype(jnp.bfloat16)

    xb = x_ref[...].astype(jnp.bfloat16)
    acc = jnp.dot(xb, w_bf_ref[...], preferred_element_type=jnp.float32)
    o_ref[...] = acc + b_ref[...]


def kernel(x, wt, b2d):
    n_out = wt.shape[1]
    *lead, K = x.shape
    x2d = x.reshape(-1, K)
    M = x2d.shape[0]
    Kp, Np = wt.shape

    tm = 512
    Mp = (M + tm - 1) // tm * tm
    if Mp != M or Kp != K:
        x2d = jnp.pad(x2d, ((0, Mp - M), (0, Kp - K)))

    grid = (Mp // tm,)

    cost = pl.CostEstimate(
        flops=2 * Mp * Np * Kp,
        transcendentals=0,
        bytes_accessed=Mp * Kp * 4 + Kp * Np * 4 + Np * 4 + Mp * Np * 4,
    )

    out = pl.pallas_call(
        _matmul_bias_kernel,
        out_shape=jax.ShapeDtypeStruct((Mp, Np), x.dtype),
        grid=grid,
        in_specs=[
            pl.BlockSpec((tm, Kp), lambda i: (i, 0)),
            pl.BlockSpec((Kp, Np), lambda i: (0, 0)),
            pl.BlockSpec((1, Np), lambda i: (0, 0)),
        ],
        out_specs=pl.BlockSpec((tm, Np), lambda i: (i, 0)),
        scratch_shapes=[pltpu.VMEM((Kp, Np), jnp.bfloat16)],
        compiler_params=pltpu.CompilerParams(
            dimension_semantics=("arbitrary",),
            vmem_limit_bytes=100 * 1024 * 1024,
        ),
        cost_estimate=cost,
    )(x2d, wt, b2d)

    if Mp != M or Np != n_out:
        out = out[:M, :n_out]
    return out.reshape(*lead, n_out)
